# Initial kernel scaffold; baseline (speedup 1.0000x reference)
import jax
import jax.numpy as jnp
from jax.experimental import pallas as pl


def kernel(node_features, edge_index, Wp, bp, Wc0, bc0, Wc1, bc1, Wc2, bc2, Wg0, bg0, Wg1, bg1, Wg2, bg2, Wm0, bm0, Wm1, bm1, Wm2, bm2):
    raise NotImplementedError("write your pallas kernel here")


# speedup vs baseline: 3.9082x; 3.9082x over previous
import functools

import jax
import jax.numpy as jnp
from jax import lax
from jax.experimental import pallas as pl
from jax.experimental.pallas import tpu as pltpu
from jax.experimental.pallas import tpu_sc as plsc

N = 10000
E = 320000
D = 128

NC = 2
NS = 16
NW = NC * NS
EPT = E // NW
CH = 80
NCH = EPT // CH
NP = 10240
RPT = NP // NS
ZR = 128

NH = 10240
HPT = NH // NS

_mesh = plsc.VectorSubcoreMesh(core_axis_name="c", subcore_axis_name="s")


@functools.partial(
    pl.kernel,
    out_type=jax.ShapeDtypeStruct((2 * NC * NH,), jnp.float32),
    mesh=_mesh,
    scratch_types=[
        pltpu.VMEM((NCH, CH), jnp.int32),
        pltpu.VMEM((NCH, CH), jnp.int32),
        pltpu.VMEM((CH,), jnp.float32),
        pltpu.VMEM((HPT,), jnp.float32),
        pltpu.VMEM_SHARED((NH,), jnp.float32),
        pltpu.VMEM_SHARED((NH,), jnp.float32),
    ],
)
def _deg_kernel(src_hbm, dst_hbm, out_hbm, src_v, dst_v, ones_v, zeros_v,
                hsrc_sh, hdst_sh):
    c = lax.axis_index("c")
    s = lax.axis_index("s")
    wid = s * NC + c

    pltpu.sync_copy(src_hbm.at[wid], src_v)
    pltpu.sync_copy(dst_hbm.at[wid], dst_v)

    for i in range(CH // 16):
        ones_v[pl.ds(i * 16, 16)] = jnp.ones((16,), jnp.float32)

    def _zero(i, _):
        zeros_v[pl.ds(i * 16, 16)] = jnp.zeros((16,), jnp.float32)
        return 0
    lax.fori_loop(0, HPT // 16, _zero, 0)

    hoff = pl.multiple_of(s * HPT, 128)
    pltpu.sync_copy(zeros_v, hsrc_sh.at[pl.ds(hoff, HPT)])
    pltpu.sync_copy(zeros_v, hdst_sh.at[pl.ds(hoff, HPT)])
    plsc.subcore_barrier()

    def _accum(i, _):
        pltpu.sync_copy(ones_v, hsrc_sh.at[src_v.at[i]], add=True)
        pltpu.sync_copy(ones_v, hdst_sh.at[dst_v.at[i]], add=True)
        return 0
    lax.fori_loop(0, NCH, _accum, 0)

    plsc.subcore_barrier()
    osrc = pl.multiple_of(c * (2 * NH) + s * HPT, 128)
    odst = pl.multiple_of(c * (2 * NH) + NH + s * HPT, 128)
    pltpu.sync_copy(hsrc_sh.at[pl.ds(hoff, HPT)], out_hbm.at[pl.ds(osrc, HPT)])
    pltpu.sync_copy(hdst_sh.at[pl.ds(hoff, HPT)], out_hbm.at[pl.ds(odst, HPT)])


HALF = NP // NC
AR = 6400
ART = AR // NS
WBT = HALF // NS
ESH = E // NS
NCH2 = ESH // CH


@functools.partial(
    pl.kernel,
    out_type=jax.ShapeDtypeStruct((NP, D), jnp.float32),
    mesh=_mesh,
    scratch_types=[
        pltpu.VMEM((NCH2, CH), jnp.int32),
        pltpu.VMEM((NCH2, CH), jnp.int32),
        pltpu.VMEM((8, CH), jnp.int32),
        pltpu.VMEM((CH, D), jnp.float32),
        pltpu.VMEM_SHARED((AR, D), jnp.float32),
        pltpu.SemaphoreType.DMA,
    ],
)
def _seg_kernel(hs_hbm, src_hbm, dst_hbm, out_hbm, src_v, dst_v, dloc_v,
                rows_v, acc_sh, sem):
    c = lax.axis_index("c")
    s = lax.axis_index("s")
    base = c * HALF

    pltpu.sync_copy(src_hbm.at[s], src_v)
    pltpu.sync_copy(dst_hbm.at[s], dst_v)

    def _zrow(i, _):
        for j in range(D // 16):
            rows_v[i, pl.ds(j * 16, 16)] = jnp.zeros((16,), jnp.float32)
        return 0
    lax.fori_loop(0, CH, _zrow, 0)

    for k in range(ART // CH):
        zoff = pl.multiple_of(s * ART + k * CH, 8)
        pltpu.sync_copy(rows_v, acc_sh.at[pl.ds(zoff, CH)])
    plsc.subcore_barrier()

    def _edge_chunk(i, _):
        gather = pltpu.async_copy(hs_hbm.at[src_v.at[i]], rows_v, sem)
        for j in range(CH // 16):
            d = dst_v[i, pl.ds(j * 16, 16)]
            l = d - base
            ok = (l >= 0) & (l < HALF)
            dloc_v[0, pl.ds(j * 16, 16)] = jnp.where(ok, l, HALF)
        gather.wait()
        pltpu.sync_copy(rows_v, acc_sh.at[dloc_v.at[0]], add=True)
        return 0
    lax.fori_loop(0, NCH2, _edge_chunk, 0)

    plsc.subcore_barrier()
    roff = pl.multiple_of(s * WBT, 8)
    ooff = pl.multiple_of(c * HALF + s * WBT, 8)
    pltpu.sync_copy(acc_sh.at[pl.ds(roff, WBT)], out_hbm.at[pl.ds(ooff, WBT)])


def _norm_body(deg_ref, nout_ref, nin_ref):
    deg = deg_ref[...]
    dsrc = deg[0:1] + deg[2:3]
    ddst = deg[1:2] + deg[3:4]
    nout_ref[...] = lax.rsqrt(jnp.clip(dsrc, 1.0, None))
    nin_ref[...] = lax.rsqrt(jnp.clip(ddst, 1.0, None))


_norm_call = pl.pallas_call(
    _norm_body,
    out_shape=[
        jax.ShapeDtypeStruct((1, NH), jnp.float32),
        jax.ShapeDtypeStruct((1, NH), jnp.float32),
    ],
)


def _proj_body(x_ref, wp_ref, bp_ref, nout_ref, h_ref, hs_ref, hg_ref):
    h = jnp.dot(x_ref[...], wp_ref[...],
                preferred_element_type=jnp.float32) + bp_ref[...]
    h_ref[...] = h
    hg_ref[...] = jnp.sum(h, axis=0, keepdims=True)
    hs_ref[...] = h * nout_ref[...]


_proj_call = pl.pallas_call(
    _proj_body,
    out_shape=[
        jax.ShapeDtypeStruct((N, D), jnp.float32),
        jax.ShapeDtypeStruct((N, D), jnp.float32),
        jax.ShapeDtypeStruct((1, D), jnp.float32),
    ],
)


def _layer_body(h_ref, mp_ref, nin_ref, nout_ref, wc_ref, bc_ref, wg_ref,
                bg_ref, hgin_ref, hnew_ref, hsnew_ref, hgout_ref):
    m = mp_ref[...] * nin_ref[...]
    conv = jnp.dot(m, wc_ref[...],
                   preferred_element_type=jnp.float32) + bc_ref[...]
    x = h_ref[...] + conv
    mu = jnp.mean(x, axis=-1, keepdims=True)
    xc = x - mu
    var = jnp.mean(xc * xc, axis=-1, keepdims=True)
    hn = xc * lax.rsqrt(var + 1e-5)
    hnew_ref[...] = hn
    hsnew_ref[...] = hn * nout_ref[...]
    hgi = jnp.sum(hn, axis=0, keepdims=True)
    g = jnp.dot(hgi, wg_ref[...],
                preferred_element_type=jnp.float32) + bg_ref[...]
    hgout_ref[...] = hgin_ref[...] + jnp.where(g >= 0, g, 0.01 * g)


_layer_call = pl.pallas_call(
    _layer_body,
    out_shape=[
        jax.ShapeDtypeStruct((N, D), jnp.float32),
        jax.ShapeDtypeStruct((N, D), jnp.float32),
        jax.ShapeDtypeStruct((1, D), jnp.float32),
    ],
)


def _mlp_body(hg_ref, w0_ref, b0_ref, w1_ref, b1_ref, w2_ref, b2_ref,
              out_ref):
    x = hg_ref[...]
    x = jnp.dot(x, w0_ref[...], preferred_element_type=jnp.float32) + b0_ref[...]
    x = jnp.maximum(x, 0.0)
    x = jnp.dot(x, w1_ref[...], preferred_element_type=jnp.float32) + b1_ref[...]
    x = jnp.maximum(x, 0.0)
    out_ref[...] = jnp.dot(x, w2_ref[...],
                           preferred_element_type=jnp.float32) + b2_ref[...]


_mlp_call = pl.pallas_call(
    _mlp_body,
    out_shape=jax.ShapeDtypeStruct((1, D), jnp.float32),
)


def kernel(node_features, edge_index, Wp, bp, Wc0, bc0, Wc1, bc1, Wc2, bc2,
           Wg0, bg0, Wg1, bg1, Wg2, bg2, Wm0, bm0, Wm1, bm1, Wm2, bm2):
    src = edge_index[0].reshape(NW, NCH, CH)
    dst = edge_index[1].reshape(NW, NCH, CH)
    src16 = edge_index[0].reshape(NS, NCH2, CH)
    dst16 = edge_index[1].reshape(NS, NCH2, CH)

    deg4 = _deg_kernel(src, dst).reshape(2 * NC, NH)

    nout_row, nin_row = _norm_call(deg4)
    nout = nout_row.reshape(NH, 1)[:N]
    nin = nin_row.reshape(NH, 1)[:N]

    h, hs, hg = _proj_call(node_features, Wp, bp.reshape(1, D), nout)

    for Wc, bc, Wg, bg in ((Wc0, bc0, Wg0, bg0),
                           (Wc1, bc1, Wg1, bg1),
                           (Wc2, bc2, Wg2, bg2)):
        mp = _seg_kernel(hs, src16, dst16)[:N]
        h, hs, hg = _layer_call(h, mp, nin, nout, Wc, bc.reshape(1, D),
                                Wg, bg.reshape(1, D), hg)

    return _mlp_call(hg, Wm0, bm0.reshape(1, D), Wm1, bm1.reshape(1, D),
                     Wm2, bm2.reshape(1, D))


# speedup vs baseline: 5.7297x; 1.4661x over previous
import functools

import jax
import jax.numpy as jnp
from jax import lax
from jax.experimental import pallas as pl
from jax.experimental.pallas import tpu as pltpu
from jax.experimental.pallas import tpu_sc as plsc

N = 10000
E = 320000
D = 128

NC = 2
NS = 16
NW = NC * NS
EPT = E // NW
CH = 80
NCH = EPT // CH
NP = 10240
RPT = NP // NS
ZR = 128

NH = 10240
HPT = NH // NS

_mesh = plsc.VectorSubcoreMesh(core_axis_name="c", subcore_axis_name="s")


@functools.partial(
    pl.kernel,
    out_type=jax.ShapeDtypeStruct((2 * NC * NH,), jnp.float32),
    mesh=_mesh,
    scratch_types=[
        pltpu.VMEM((NCH, CH), jnp.int32),
        pltpu.VMEM((NCH, CH), jnp.int32),
        pltpu.VMEM((CH,), jnp.float32),
        pltpu.VMEM((HPT,), jnp.float32),
        pltpu.VMEM_SHARED((NH,), jnp.float32),
        pltpu.VMEM_SHARED((NH,), jnp.float32),
    ],
)
def _deg_kernel(src_hbm, dst_hbm, out_hbm, src_v, dst_v, ones_v, zeros_v,
                hsrc_sh, hdst_sh):
    c = lax.axis_index("c")
    s = lax.axis_index("s")
    wid = s * NC + c

    pltpu.sync_copy(src_hbm.at[wid], src_v)
    pltpu.sync_copy(dst_hbm.at[wid], dst_v)

    for i in range(CH // 16):
        ones_v[pl.ds(i * 16, 16)] = jnp.ones((16,), jnp.float32)

    def _zero(i, _):
        zeros_v[pl.ds(i * 16, 16)] = jnp.zeros((16,), jnp.float32)
        return 0
    lax.fori_loop(0, HPT // 16, _zero, 0)

    hoff = pl.multiple_of(s * HPT, 128)
    pltpu.sync_copy(zeros_v, hsrc_sh.at[pl.ds(hoff, HPT)])
    pltpu.sync_copy(zeros_v, hdst_sh.at[pl.ds(hoff, HPT)])
    plsc.subcore_barrier()

    def _accum(i, _):
        pltpu.sync_copy(ones_v, hsrc_sh.at[src_v.at[i]], add=True)
        pltpu.sync_copy(ones_v, hdst_sh.at[dst_v.at[i]], add=True)
        return 0
    lax.fori_loop(0, NCH, _accum, 0)

    plsc.subcore_barrier()
    osrc = pl.multiple_of(c * (2 * NH) + s * HPT, 128)
    odst = pl.multiple_of(c * (2 * NH) + NH + s * HPT, 128)
    pltpu.sync_copy(hsrc_sh.at[pl.ds(hoff, HPT)], out_hbm.at[pl.ds(osrc, HPT)])
    pltpu.sync_copy(hdst_sh.at[pl.ds(hoff, HPT)], out_hbm.at[pl.ds(odst, HPT)])


HALF = NP // NC
AR = 5248
ART = AR // NS
WBT = HALF // NS
ESH = E // NS
NCH2 = ESH // CH


@functools.partial(
    pl.kernel,
    out_type=jax.ShapeDtypeStruct((NP, D), jnp.float32),
    mesh=_mesh,
    scratch_types=[
        pltpu.VMEM((NCH2, CH), jnp.int32),
        pltpu.VMEM((NCH2, CH), jnp.int32),
        pltpu.VMEM((8, CH), jnp.int32),
        pltpu.VMEM((CH, D), jnp.float32),
        pltpu.VMEM((CH, D), jnp.float32),
        pltpu.VMEM_SHARED((AR, D), jnp.float32),
        pltpu.SemaphoreType.DMA,
        pltpu.SemaphoreType.DMA,
    ],
)
def _seg_kernel(hs_hbm, src_hbm, dst_hbm, out_hbm, src_v, dst_v, dloc_v,
                rows0_v, rows1_v, acc_sh, sem0, sem1):
    c = lax.axis_index("c")
    s = lax.axis_index("s")
    base = c * HALF

    pltpu.sync_copy(src_hbm.at[s], src_v)
    pltpu.sync_copy(dst_hbm.at[s], dst_v)

    def _zrow(i, _):
        for j in range(D // 16):
            rows0_v[i, pl.ds(j * 16, 16)] = jnp.zeros((16,), jnp.float32)
        return 0
    lax.fori_loop(0, CH, _zrow, 0)

    for k in range(ART // CH):
        zoff = pl.multiple_of(s * ART + k * CH, 8)
        pltpu.sync_copy(rows0_v, acc_sh.at[pl.ds(zoff, CH)])
    ztail = pl.multiple_of(s * ART + (ART // CH) * CH, 8)
    pltpu.sync_copy(rows0_v.at[pl.ds(0, ART - (ART // CH) * CH)],
                    acc_sh.at[pl.ds(ztail, ART - (ART // CH) * CH)])
    plsc.subcore_barrier()

    def _transform(i, row):
        for j in range(CH // 16):
            d = dst_v[i, pl.ds(j * 16, 16)]
            l = d - base
            ok = (l >= 0) & (l < HALF)
            dloc_v[row, pl.ds(j * 16, 16)] = jnp.where(ok, l, HALF)

    pltpu.async_copy(hs_hbm.at[src_v.at[0]], rows0_v, sem0)

    def _pair(p, _):
        i0 = p * 2
        i1 = i0 + 1
        pltpu.async_copy(hs_hbm.at[src_v.at[i1]], rows1_v, sem1)
        _transform(i0, 0)
        pltpu.make_async_copy(hs_hbm.at[src_v.at[i0]], rows0_v, sem0).wait()
        pltpu.sync_copy(rows0_v, acc_sh.at[dloc_v.at[0]], add=True)

        @pl.when(i0 + 2 < NCH2)
        def _():
            pltpu.async_copy(hs_hbm.at[src_v.at[i0 + 2]], rows0_v, sem0)

        _transform(i1, 1)
        pltpu.make_async_copy(hs_hbm.at[src_v.at[i1]], rows1_v, sem1).wait()
        pltpu.sync_copy(rows1_v, acc_sh.at[dloc_v.at[1]], add=True)
        return 0
    lax.fori_loop(0, NCH2 // 2, _pair, 0)

    plsc.subcore_barrier()
    roff = pl.multiple_of(s * WBT, 8)
    ooff = pl.multiple_of(c * HALF + s * WBT, 8)
    pltpu.sync_copy(acc_sh.at[pl.ds(roff, WBT)], out_hbm.at[pl.ds(ooff, WBT)])


def _norm_body(deg_ref, nout_ref, nin_ref):
    deg = deg_ref[...]
    dsrc = deg[0:1] + deg[2:3]
    ddst = deg[1:2] + deg[3:4]
    nout_ref[...] = lax.rsqrt(jnp.clip(dsrc, 1.0, None))
    nin_ref[...] = lax.rsqrt(jnp.clip(ddst, 1.0, None))


_norm_call = pl.pallas_call(
    _norm_body,
    out_shape=[
        jax.ShapeDtypeStruct((1, NH), jnp.float32),
        jax.ShapeDtypeStruct((1, NH), jnp.float32),
    ],
)


def _proj_body(x_ref, wp_ref, bp_ref, nout_ref, h_ref, hs_ref, hg_ref):
    h = jnp.dot(x_ref[...], wp_ref[...],
                preferred_element_type=jnp.float32) + bp_ref[...]
    h_ref[...] = h
    hg_ref[...] = jnp.sum(h, axis=0, keepdims=True)
    hs_ref[...] = h * nout_ref[...]


_proj_call = pl.pallas_call(
    _proj_body,
    out_shape=[
        jax.ShapeDtypeStruct((N, D), jnp.float32),
        jax.ShapeDtypeStruct((N, D), jnp.float32),
        jax.ShapeDtypeStruct((1, D), jnp.float32),
    ],
)


def _layer_body(h_ref, mp_ref, nin_ref, nout_ref, wc_ref, bc_ref, wg_ref,
                bg_ref, hgin_ref, hnew_ref, hsnew_ref, hgout_ref):
    m = mp_ref[...] * nin_ref[...]
    conv = jnp.dot(m, wc_ref[...],
                   preferred_element_type=jnp.float32) + bc_ref[...]
    x = h_ref[...] + conv
    mu = jnp.mean(x, axis=-1, keepdims=True)
    xc = x - mu
    var = jnp.mean(xc * xc, axis=-1, keepdims=True)
    hn = xc * lax.rsqrt(var + 1e-5)
    hnew_ref[...] = hn
    hsnew_ref[...] = hn * nout_ref[...]
    hgi = jnp.sum(hn, axis=0, keepdims=True)
    g = jnp.dot(hgi, wg_ref[...],
                preferred_element_type=jnp.float32) + bg_ref[...]
    hgout_ref[...] = hgin_ref[...] + jnp.where(g >= 0, g, 0.01 * g)


_layer_call = pl.pallas_call(
    _layer_body,
    out_shape=[
        jax.ShapeDtypeStruct((N, D), jnp.float32),
        jax.ShapeDtypeStruct((N, D), jnp.float32),
        jax.ShapeDtypeStruct((1, D), jnp.float32),
    ],
)


def _mlp_body(hg_ref, w0_ref, b0_ref, w1_ref, b1_ref, w2_ref, b2_ref,
              out_ref):
    x = hg_ref[...]
    x = jnp.dot(x, w0_ref[...], preferred_element_type=jnp.float32) + b0_ref[...]
    x = jnp.maximum(x, 0.0)
    x = jnp.dot(x, w1_ref[...], preferred_element_type=jnp.float32) + b1_ref[...]
    x = jnp.maximum(x, 0.0)
    out_ref[...] = jnp.dot(x, w2_ref[...],
                           preferred_element_type=jnp.float32) + b2_ref[...]


_mlp_call = pl.pallas_call(
    _mlp_body,
    out_shape=jax.ShapeDtypeStruct((1, D), jnp.float32),
)


def kernel(node_features, edge_index, Wp, bp, Wc0, bc0, Wc1, bc1, Wc2, bc2,
           Wg0, bg0, Wg1, bg1, Wg2, bg2, Wm0, bm0, Wm1, bm1, Wm2, bm2):
    src = edge_index[0].reshape(NW, NCH, CH)
    dst = edge_index[1].reshape(NW, NCH, CH)
    src16 = edge_index[0].reshape(NS, NCH2, CH)
    dst16 = edge_index[1].reshape(NS, NCH2, CH)

    deg4 = _deg_kernel(src, dst).reshape(2 * NC, NH)

    nout_row, nin_row = _norm_call(deg4)
    nout = nout_row.reshape(NH, 1)[:N]
    nin = nin_row.reshape(NH, 1)[:N]

    h, hs, hg = _proj_call(node_features, Wp, bp.reshape(1, D), nout)

    for Wc, bc, Wg, bg in ((Wc0, bc0, Wg0, bg0),
                           (Wc1, bc1, Wg1, bg1),
                           (Wc2, bc2, Wg2, bg2)):
        mp = _seg_kernel(hs, src16, dst16)[:N]
        h, hs, hg = _layer_call(h, mp, nin, nout, Wc, bc.reshape(1, D),
                                Wg, bg.reshape(1, D), hg)

    return _mlp_call(hg, Wm0, bm0.reshape(1, D), Wm1, bm1.reshape(1, D),
                     Wm2, bm2.reshape(1, D))
